# baseline (speedup 1.0000x reference)
import functools

import jax
import jax.numpy as jnp
from jax import lax
from jax.experimental import pallas as pl
from jax.experimental.pallas import tpu as pltpu
from jax.experimental.pallas import tpu_sc as plsc

NN = 10000
EE = 320000
FIN = 128
FH = 256
NC = 2
NS = 16
CH = 32
NB = 4
PC = 16
K1 = 320
K2 = 2 * K1
EPAD = NC * NS * K1 * CH
NPAD = 10112
RPT = NPAD // NS


@functools.cache
def _mesh():
    return plsc.VectorSubcoreMesh(core_axis_name="c", subcore_axis_name="s",
                                  num_cores=NC, num_subcores=NS)


def _zero_buf(buf, nrows):
    def z(t, _):
        buf[t // 8, pl.ds((t % 8) * 16, 16)] = jnp.zeros((16,), jnp.float32)
        return 0
    lax.fori_loop(0, nrows * 8, z, 0)


def _chunks():
    out = []
    o = 0
    while o < RPT:
        out.append((o, min(CH, RPT - o)))
        o += CH
    return out


def _zero_acc_slice(buf, acc_sh, tbase):
    for o, c in _chunks():
        pltpu.sync_copy(buf.at[pl.ds(0, c)], acc_sh.at[pl.ds(tbase + o, c)])


def _dump_acc_slice(bufa, bufb, sema, semb, acc_sh, tbase, out_hbm):
    bufs = (bufa, bufb)
    sems = (sema, semb)
    pend = [None, None]
    for i, (o, c) in enumerate(_chunks()):
        b = i % 2
        if pend[b] is not None:
            pend[b].wait()
        pltpu.sync_copy(acc_sh.at[pl.ds(tbase + o, c)], bufs[b].at[pl.ds(0, c)])
        pend[b] = pltpu.async_copy(bufs[b].at[pl.ds(0, c)],
                                   out_hbm.at[pl.ds(tbase + o, c)], sems[b])
    for b in range(2):
        if pend[b] is not None:
            pend[b].wait()


def _sc_agg1_body(x_hbm, srcp_hbm, dstp_hbm, p_out, deg_out,
                  sidx, didx, b0, b1, b2, b3, degtile, acc_sh,
                  g0s, g1s, g2s, g3s, s0s, s1s, s2s, s3s):
    cid = lax.axis_index("c")
    sid = lax.axis_index("s")
    wid = sid * NC + cid
    tbase = sid * RPT

    _zero_buf(b0, CH)
    _zero_buf(b1, CH)

    def zdeg(t, _):
        degtile[pl.ds(t * 16, 16)] = jnp.zeros((16,), jnp.float32)
        return 0
    lax.fori_loop(0, NPAD // 16, zdeg, 0)

    _zero_acc_slice(b0, acc_sh, tbase)
    plsc.subcore_barrier()

    ones16 = jnp.ones((16,), jnp.float32)
    bufs = (b0, b1, b2, b3)
    gsem = (g0s, g1s, g2s, g3s)
    ssem = (s0s, s1s, s2s, s3s)

    def one_pass(hp, _):
        base = wid * K1 + hp * PC
        pltpu.sync_copy(srcp_hbm.at[pl.ds(base, PC)], sidx)
        pltpu.sync_copy(dstp_hbm.at[pl.ds(base, PC)], didx)
        gd = [pltpu.async_copy(x_hbm.at[sidx.at[k]], bufs[k], gsem[k])
              for k in range(NB)]
        sd = [None] * NB
        for c in range(PC):
            b = c % NB
            gd[b].wait()
            sd[b] = pltpu.async_copy(bufs[b], acc_sh.at[didx.at[c]],
                                     ssem[b], add=True)
            for j in range(CH // 16):
                d = didx[c, pl.ds(j * 16, 16)]
                plsc.addupdate_scatter(degtile, [d], ones16)
            n = c + 2
            if n >= NB and n < PC:
                bn = n % NB
                sd[bn].wait()
                gd[bn] = pltpu.async_copy(x_hbm.at[sidx.at[n]],
                                          bufs[bn], gsem[bn])
        for k in range(PC - NB, PC):
            sd[k % NB].wait()
        return 0
    lax.fori_loop(0, K1 // PC, one_pass, 0)

    plsc.subcore_barrier()
    _dump_acc_slice(b0, b1, g0s, g1s, acc_sh, tbase, p_out.at[cid])
    pltpu.sync_copy(degtile, deg_out.at[cid, sid])


@functools.cache
def _sc_agg1():
  return pl.kernel(
    _sc_agg1_body,
    out_type=[jax.ShapeDtypeStruct((NC, NPAD, FIN), jnp.float32),
              jax.ShapeDtypeStruct((NC, NS, NPAD), jnp.float32)],
    mesh=_mesh(),
    compiler_params=pltpu.CompilerParams(needs_layout_passes=False),
    scratch_types=[
        pltpu.VMEM((PC, CH), jnp.int32),
        pltpu.VMEM((PC, CH), jnp.int32),
        pltpu.VMEM((CH, FIN), jnp.float32),
        pltpu.VMEM((CH, FIN), jnp.float32),
        pltpu.VMEM((CH, FIN), jnp.float32),
        pltpu.VMEM((CH, FIN), jnp.float32),
        pltpu.VMEM((NPAD,), jnp.float32),
        pltpu.VMEM_SHARED((NPAD, FIN), jnp.float32),
        pltpu.SemaphoreType.DMA,
        pltpu.SemaphoreType.DMA,
        pltpu.SemaphoreType.DMA,
        pltpu.SemaphoreType.DMA,
        pltpu.SemaphoreType.DMA,
        pltpu.SemaphoreType.DMA,
        pltpu.SemaphoreType.DMA,
        pltpu.SemaphoreType.DMA,
    ],
  )


def _sc_agg2_body(h2v_hbm, srcp_hbm, dstp_hbm, a_out,
                  sidx, didx, sc0, sc1, sc2, sc3, b0, b1, b2, b3, acc_sh,
                  g0s, g1s, g2s, g3s, s0s, s1s, s2s, s3s):
    cid = lax.axis_index("c")
    sid = lax.axis_index("s")
    tbase = sid * RPT

    _zero_buf(b0, CH)
    _zero_buf(b1, CH)
    _zero_acc_slice(b0, acc_sh, tbase)
    plsc.subcore_barrier()

    bufs = (b0, b1, b2, b3)
    scx = (sc0, sc1, sc2, sc3)
    gsem = (g0s, g1s, g2s, g3s)
    ssem = (s0s, s1s, s2s, s3s)

    def scale_into(dst, c):
        def scale(j, _):
            v = sidx[c, pl.ds(j * 16, 16)]
            dst[pl.ds(j * 16, 16)] = v * 2 + cid
            return 0
        lax.fori_loop(0, CH // 16, scale, 0)

    def one_pass(hp, _):
        base = sid * K2 + hp * PC
        pltpu.sync_copy(srcp_hbm.at[pl.ds(base, PC)], sidx)
        pltpu.sync_copy(dstp_hbm.at[pl.ds(base, PC)], didx)
        gd = [None] * NB
        for k in range(NB):
            scale_into(scx[k], k)
            gd[k] = pltpu.async_copy(h2v_hbm.at[scx[k]], bufs[k], gsem[k])
        sd = [None] * NB
        for c in range(PC):
            b = c % NB
            gd[b].wait()
            sd[b] = pltpu.async_copy(bufs[b], acc_sh.at[didx.at[c]],
                                     ssem[b], add=True)
            n = c + 2
            if n >= NB and n < PC:
                bn = n % NB
                sd[bn].wait()
                scale_into(scx[bn], n)
                gd[bn] = pltpu.async_copy(h2v_hbm.at[scx[bn]],
                                          bufs[bn], gsem[bn])
        for k in range(PC - NB, PC):
            sd[k % NB].wait()
        return 0
    lax.fori_loop(0, K2 // PC, one_pass, 0)

    plsc.subcore_barrier()
    _dump_acc_slice(b0, b1, g0s, g1s, acc_sh, tbase, a_out.at[cid])


@functools.cache
def _sc_agg2():
  return pl.kernel(
    _sc_agg2_body,
    out_type=jax.ShapeDtypeStruct((NC, NPAD, FIN), jnp.float32),
    mesh=_mesh(),
    compiler_params=pltpu.CompilerParams(needs_layout_passes=False),
    scratch_types=[
        pltpu.VMEM((PC, CH), jnp.int32),
        pltpu.VMEM((PC, CH), jnp.int32),
        pltpu.VMEM((CH,), jnp.int32),
        pltpu.VMEM((CH,), jnp.int32),
        pltpu.VMEM((CH,), jnp.int32),
        pltpu.VMEM((CH,), jnp.int32),
        pltpu.VMEM((CH, FIN), jnp.float32),
        pltpu.VMEM((CH, FIN), jnp.float32),
        pltpu.VMEM((CH, FIN), jnp.float32),
        pltpu.VMEM((CH, FIN), jnp.float32),
        pltpu.VMEM_SHARED((NPAD, FIN), jnp.float32),
        pltpu.SemaphoreType.DMA,
        pltpu.SemaphoreType.DMA,
        pltpu.SemaphoreType.DMA,
        pltpu.SemaphoreType.DMA,
        pltpu.SemaphoreType.DMA,
        pltpu.SemaphoreType.DMA,
        pltpu.SemaphoreType.DMA,
        pltpu.SemaphoreType.DMA,
    ],
  )


BN = 2048


def _rdeg(d_ref):
    deg = jnp.sum(d_ref[...], axis=(0, 1))[:, None]
    return 1.0 / jnp.maximum(deg, 1.0)


def _tc1_body(x_ref, p0_ref, p1_ref, d_ref, wl_ref, wr_ref, b_ref, h_ref):
    agg = (p0_ref[0] + p1_ref[0]) * _rdeg(d_ref)
    h = jnp.dot(agg, wl_ref[...], preferred_element_type=jnp.float32)
    h = h + jnp.dot(x_ref[...], wr_ref[...], preferred_element_type=jnp.float32)
    h = h + b_ref[...]
    h_ref[...] = jnp.maximum(h, 0.0)


_tc1 = pl.pallas_call(
    _tc1_body,
    grid=(pl.cdiv(NN, BN),),
    in_specs=[
        pl.BlockSpec((BN, FIN), lambda i: (i, 0)),
        pl.BlockSpec((1, BN, FIN), lambda i: (0, i, 0)),
        pl.BlockSpec((1, BN, FIN), lambda i: (1, i, 0)),
        pl.BlockSpec((NC, NS, BN), lambda i: (0, 0, i)),
        pl.BlockSpec((FIN, FH), lambda i: (0, 0)),
        pl.BlockSpec((FIN, FH), lambda i: (0, 0)),
        pl.BlockSpec((1, FH), lambda i: (0, 0)),
    ],
    out_specs=pl.BlockSpec((BN, FH), lambda i: (i, 0)),
    out_shape=jax.ShapeDtypeStruct((NN, FH), jnp.float32),
)


def _tc2_body(h_ref, a0_ref, a1_ref, d_ref, wl0_ref, wl1_ref,
              wr_ref, b_ref, wsy_ref, h2_ref, osy_ref):
    rdeg = _rdeg(d_ref)
    h2 = jnp.dot(a0_ref[0] * rdeg, wl0_ref[...],
                 preferred_element_type=jnp.float32)
    h2 = h2 + jnp.dot(a1_ref[0] * rdeg, wl1_ref[...],
                      preferred_element_type=jnp.float32)
    h2 = h2 + jnp.dot(h_ref[...], wr_ref[...],
                      preferred_element_type=jnp.float32)
    h2 = h2 + b_ref[...]
    h2_ref[...] = h2
    osy_ref[...] = jnp.dot(h2, wsy_ref[...], preferred_element_type=jnp.float32)


_tc2 = pl.pallas_call(
    _tc2_body,
    grid=(pl.cdiv(NN, BN),),
    in_specs=[
        pl.BlockSpec((BN, FH), lambda i: (i, 0)),
        pl.BlockSpec((1, BN, FIN), lambda i: (0, i, 0)),
        pl.BlockSpec((1, BN, FIN), lambda i: (1, i, 0)),
        pl.BlockSpec((NC, NS, BN), lambda i: (0, 0, i)),
        pl.BlockSpec((FIN, FH), lambda i: (0, 0)),
        pl.BlockSpec((FIN, FH), lambda i: (0, 0)),
        pl.BlockSpec((FH, FH), lambda i: (0, 0)),
        pl.BlockSpec((1, FH), lambda i: (0, 0)),
        pl.BlockSpec((FH, FIN), lambda i: (0, 0)),
    ],
    out_specs=[
        pl.BlockSpec((BN, FH), lambda i: (i, 0)),
        pl.BlockSpec((BN, FIN), lambda i: (i, 0)),
    ],
    out_shape=[
        jax.ShapeDtypeStruct((NN, FH), jnp.float32),
        jax.ShapeDtypeStruct((NN, FIN), jnp.float32),
    ],
)


def kernel(x, edge_index, W1l, b1, W1r, W2l, b2, W2r, Ws, Wy):
    src = edge_index[0]
    dst = edge_index[1]
    pad = EPAD - EE
    srcp = jnp.concatenate([src, jnp.zeros((pad,), jnp.int32)]
                           ).reshape(EPAD // CH, CH)
    dstp = jnp.concatenate([dst, jnp.full((pad,), NN, jnp.int32)]
                           ).reshape(EPAD // CH, CH)

    p, degp = _sc_agg1()(x, srcp, dstp)
    h = _tc1(x, p, p, degp, W1l.T, W1r.T, b1.reshape(1, FH))

    a2 = _sc_agg2()(h.reshape(2 * NN, FIN), srcp, dstp)

    wsy = jnp.concatenate([Ws, Wy], axis=0)
    wsy_pad = jnp.pad(wsy, ((0, FIN - wsy.shape[0]), (0, 0))).T
    w2lt = W2l.T
    h2, osy = _tc2(h, a2, a2, degp, w2lt[:FIN], w2lt[FIN:], W2r.T,
                   b2.reshape(1, FH), wsy_pad)
    return osy[:, :2], osy[:, 2:42], h2


# speedup vs baseline: 1.0711x; 1.0711x over previous
import functools

import jax
import jax.numpy as jnp
from jax import lax
from jax.experimental import pallas as pl
from jax.experimental.pallas import tpu as pltpu
from jax.experimental.pallas import tpu_sc as plsc

NN = 10000
EE = 320000
FIN = 128
FH = 256
NC = 2
NS = 16
PC = 16
CH1, NB1 = 64, 2
CH2, NB2 = 64, 2
EPAD = 327680
K1 = EPAD // (NC * NS) // CH1
K2 = EPAD // NS // CH2
NPAD = 10112
RPT = NPAD // NS
ZCH = 64


@functools.cache
def _mesh():
    return plsc.VectorSubcoreMesh(core_axis_name="c", subcore_axis_name="s",
                                  num_cores=NC, num_subcores=NS)


def _zero_buf(buf, nrows):
    def z(t, _):
        buf[t // 8, pl.ds((t % 8) * 16, 16)] = jnp.zeros((16,), jnp.float32)
        return 0
    lax.fori_loop(0, nrows * 8, z, 0)


def _chunks(step):
    out = []
    o = 0
    while o < RPT:
        out.append((o, min(step, RPT - o)))
        o += step
    return out


def _zero_acc_slice(buf, acc_sh, tbase, step):
    for o, c in _chunks(step):
        pltpu.sync_copy(buf.at[pl.ds(0, c)], acc_sh.at[pl.ds(tbase + o, c)])


def _dump_acc_slice(bufa, bufb, sema, semb, acc_sh, tbase, out_hbm, step):
    bufs = (bufa, bufb)
    sems = (sema, semb)
    pend = [None, None]
    for i, (o, c) in enumerate(_chunks(step)):
        b = i % 2
        if pend[b] is not None:
            pend[b].wait()
        pltpu.sync_copy(acc_sh.at[pl.ds(tbase + o, c)], bufs[b].at[pl.ds(0, c)])
        pend[b] = pltpu.async_copy(bufs[b].at[pl.ds(0, c)],
                                   out_hbm.at[pl.ds(tbase + o, c)], sems[b])
    for b in range(2):
        if pend[b] is not None:
            pend[b].wait()


def _ring(PCn, NBn, wait_g, issue_s, issue_g, drain_extra=None):
    gd = [issue_g(k) for k in range(NBn)]
    sd = [None] * NBn
    for c in range(PCn):
        b = c % NBn
        gd[b].wait()
        sd[b] = issue_s(c, b)
        if drain_extra is not None:
            drain_extra(c)
        n = c + 2
        if n >= NBn and n < PCn:
            bn = n % NBn
            sd[bn].wait()
            gd[bn] = issue_g(n)
    for k in range(PCn - NBn, PCn):
        sd[k % NBn].wait()


def _sc_agg1_body(x_hbm, srcp_hbm, dstp_hbm, p_out, deg_out, *scr):
    sidx, didx = scr[0], scr[1]
    bufs = scr[2:2 + NB1]
    degtile = scr[2 + NB1]
    acc_sh = scr[3 + NB1]
    gsem = scr[4 + NB1:4 + 2 * NB1]
    ssem = scr[4 + 2 * NB1:4 + 3 * NB1]

    cid = lax.axis_index("c")
    sid = lax.axis_index("s")
    wid = sid * NC + cid
    tbase = sid * RPT

    _zero_buf(bufs[0], CH1)
    _zero_buf(bufs[1], CH1)

    def zdeg(t, _):
        degtile[pl.ds(t * 16, 16)] = jnp.zeros((16,), jnp.float32)
        return 0
    lax.fori_loop(0, NPAD // 16, zdeg, 0)

    _zero_acc_slice(bufs[0], acc_sh, tbase, CH1)
    plsc.subcore_barrier()

    ones16 = jnp.ones((16,), jnp.float32)

    def one_pass(hp, _):
        base = wid * K1 + hp * PC
        pltpu.sync_copy(srcp_hbm.at[pl.ds(base, PC)], sidx)
        pltpu.sync_copy(dstp_hbm.at[pl.ds(base, PC)], didx)

        def issue_g(c):
            return pltpu.async_copy(x_hbm.at[sidx.at[c]], bufs[c % NB1],
                                    gsem[c % NB1])

        def issue_s(c, b):
            return pltpu.async_copy(bufs[b], acc_sh.at[didx.at[c]],
                                    ssem[b], add=True)

        def deg(c):
            for j in range(CH1 // 16):
                d = didx[c, pl.ds(j * 16, 16)]
                plsc.addupdate_scatter(degtile, [d], ones16)

        _ring(PC, NB1, None, issue_s, issue_g, drain_extra=deg)
        return 0
    lax.fori_loop(0, K1 // PC, one_pass, 0)

    plsc.subcore_barrier()
    _dump_acc_slice(bufs[0], bufs[1], gsem[0], gsem[1], acc_sh, tbase,
                    p_out.at[cid], CH1)
    pltpu.sync_copy(degtile, deg_out.at[cid, sid])


@functools.cache
def _sc_agg1():
  return pl.kernel(
    _sc_agg1_body,
    out_type=[jax.ShapeDtypeStruct((NC, NPAD, FIN), jnp.float32),
              jax.ShapeDtypeStruct((NC, NS, NPAD), jnp.float32)],
    mesh=_mesh(),
    compiler_params=pltpu.CompilerParams(needs_layout_passes=False),
    scratch_types=(
        [pltpu.VMEM((PC, CH1), jnp.int32)] * 2
        + [pltpu.VMEM((CH1, FIN), jnp.float32)] * NB1
        + [pltpu.VMEM((NPAD,), jnp.float32)]
        + [pltpu.VMEM_SHARED((NPAD, FIN), jnp.float32)]
        + [pltpu.SemaphoreType.DMA] * (2 * NB1)
    ),
  )


def _sc_agg2_body(h2v_hbm, srcp_hbm, dstp_hbm, a_out, *scr):
    sidx, didx = scr[0], scr[1]
    scx = scr[2:2 + NB2]
    bufs = scr[2 + NB2:2 + 2 * NB2]
    acc_sh = scr[2 + 2 * NB2]
    gsem = scr[3 + 2 * NB2:3 + 3 * NB2]
    ssem = scr[3 + 3 * NB2:3 + 4 * NB2]

    cid = lax.axis_index("c")
    sid = lax.axis_index("s")
    tbase = sid * RPT

    _zero_buf(bufs[0], CH2)
    _zero_buf(bufs[1], CH2)
    _zero_acc_slice(bufs[0], acc_sh, tbase, CH2)
    plsc.subcore_barrier()

    def scale_into(dst, c):
        def scale(j, _):
            v = sidx[c, pl.ds(j * 16, 16)]
            dst[pl.ds(j * 16, 16)] = v * 2 + cid
            return 0
        lax.fori_loop(0, CH2 // 16, scale, 0)

    def one_pass(hp, _):
        base = sid * K2 + hp * PC
        pltpu.sync_copy(srcp_hbm.at[pl.ds(base, PC)], sidx)
        pltpu.sync_copy(dstp_hbm.at[pl.ds(base, PC)], didx)

        def issue_g(c):
            b = c % NB2
            scale_into(scx[b], c)
            return pltpu.async_copy(h2v_hbm.at[scx[b]], bufs[b], gsem[b])

        def issue_s(c, b):
            return pltpu.async_copy(bufs[b], acc_sh.at[didx.at[c]],
                                    ssem[b], add=True)

        _ring(PC, NB2, None, issue_s, issue_g)
        return 0
    lax.fori_loop(0, K2 // PC, one_pass, 0)

    plsc.subcore_barrier()
    _dump_acc_slice(bufs[0], bufs[1], gsem[0], gsem[1], acc_sh, tbase,
                    a_out.at[cid], CH2)


@functools.cache
def _sc_agg2():
  return pl.kernel(
    _sc_agg2_body,
    out_type=jax.ShapeDtypeStruct((NC, NPAD, FIN), jnp.float32),
    mesh=_mesh(),
    compiler_params=pltpu.CompilerParams(needs_layout_passes=False),
    scratch_types=(
        [pltpu.VMEM((PC, CH2), jnp.int32)] * 2
        + [pltpu.VMEM((CH2,), jnp.int32)] * NB2
        + [pltpu.VMEM((CH2, FIN), jnp.float32)] * NB2
        + [pltpu.VMEM_SHARED((NPAD, FIN), jnp.float32)]
        + [pltpu.SemaphoreType.DMA] * (2 * NB2)
    ),
  )


BN = 2048


def _rdeg(d_ref):
    deg = jnp.sum(d_ref[...], axis=(0, 1))[:, None]
    return 1.0 / jnp.maximum(deg, 1.0)


def _tc1_body(x_ref, p0_ref, p1_ref, d_ref, wl_ref, wr_ref, b_ref, h_ref):
    agg = (p0_ref[0] + p1_ref[0]) * _rdeg(d_ref)
    h = jnp.dot(agg, wl_ref[...], preferred_element_type=jnp.float32)
    h = h + jnp.dot(x_ref[...], wr_ref[...], preferred_element_type=jnp.float32)
    h = h + b_ref[...]
    h_ref[...] = jnp.maximum(h, 0.0)


_tc1 = pl.pallas_call(
    _tc1_body,
    grid=(pl.cdiv(NN, BN),),
    in_specs=[
        pl.BlockSpec((BN, FIN), lambda i: (i, 0)),
        pl.BlockSpec((1, BN, FIN), lambda i: (0, i, 0)),
        pl.BlockSpec((1, BN, FIN), lambda i: (1, i, 0)),
        pl.BlockSpec((NC, NS, BN), lambda i: (0, 0, i)),
        pl.BlockSpec((FIN, FH), lambda i: (0, 0)),
        pl.BlockSpec((FIN, FH), lambda i: (0, 0)),
        pl.BlockSpec((1, FH), lambda i: (0, 0)),
    ],
    out_specs=pl.BlockSpec((BN, FH), lambda i: (i, 0)),
    out_shape=jax.ShapeDtypeStruct((NN, FH), jnp.float32),
)


def _tc2_body(h_ref, a0_ref, a1_ref, d_ref, wl0_ref, wl1_ref,
              wr_ref, b_ref, wsy_ref, h2_ref, osy_ref):
    rdeg = _rdeg(d_ref)
    h2 = jnp.dot(a0_ref[0] * rdeg, wl0_ref[...],
                 preferred_element_type=jnp.float32)
    h2 = h2 + jnp.dot(a1_ref[0] * rdeg, wl1_ref[...],
                      preferred_element_type=jnp.float32)
    h2 = h2 + jnp.dot(h_ref[...], wr_ref[...],
                      preferred_element_type=jnp.float32)
    h2 = h2 + b_ref[...]
    h2_ref[...] = h2
    osy_ref[...] = jnp.dot(h2, wsy_ref[...], preferred_element_type=jnp.float32)


_tc2 = pl.pallas_call(
    _tc2_body,
    grid=(pl.cdiv(NN, BN),),
    in_specs=[
        pl.BlockSpec((BN, FH), lambda i: (i, 0)),
        pl.BlockSpec((1, BN, FIN), lambda i: (0, i, 0)),
        pl.BlockSpec((1, BN, FIN), lambda i: (1, i, 0)),
        pl.BlockSpec((NC, NS, BN), lambda i: (0, 0, i)),
        pl.BlockSpec((FIN, FH), lambda i: (0, 0)),
        pl.BlockSpec((FIN, FH), lambda i: (0, 0)),
        pl.BlockSpec((FH, FH), lambda i: (0, 0)),
        pl.BlockSpec((1, FH), lambda i: (0, 0)),
        pl.BlockSpec((FH, FIN), lambda i: (0, 0)),
    ],
    out_specs=[
        pl.BlockSpec((BN, FH), lambda i: (i, 0)),
        pl.BlockSpec((BN, FIN), lambda i: (i, 0)),
    ],
    out_shape=[
        jax.ShapeDtypeStruct((NN, FH), jnp.float32),
        jax.ShapeDtypeStruct((NN, FIN), jnp.float32),
    ],
)


def kernel(x, edge_index, W1l, b1, W1r, W2l, b2, W2r, Ws, Wy):
    src = edge_index[0]
    dst = edge_index[1]
    pad = EPAD - EE
    src_p = jnp.concatenate([src, jnp.zeros((pad,), jnp.int32)])
    dst_p = jnp.concatenate([dst, jnp.full((pad,), NN, jnp.int32)])
    srcp1 = src_p.reshape(EPAD // CH1, CH1)
    dstp1 = dst_p.reshape(EPAD // CH1, CH1)
    srcp2 = src_p.reshape(EPAD // CH2, CH2)
    dstp2 = dst_p.reshape(EPAD // CH2, CH2)

    p, degp = _sc_agg1()(x, srcp1, dstp1)
    h = _tc1(x, p, p, degp, W1l.T, W1r.T, b1.reshape(1, FH))

    a2 = _sc_agg2()(h.reshape(2 * NN, FIN), srcp2, dstp2)

    wsy = jnp.concatenate([Ws, Wy], axis=0)
    wsy_pad = jnp.pad(wsy, ((0, FIN - wsy.shape[0]), (0, 0))).T
    w2lt = W2l.T
    h2, osy = _tc2(h, a2, a2, degp, w2lt[:FIN], w2lt[FIN:], W2r.T,
                   b2.reshape(1, FH), wsy_pad)
    return osy[:, :2], osy[:, 2:42], h2
